# baseline (speedup 1.0000x reference)
import functools

import jax
import jax.numpy as jnp
from jax import lax
from jax.experimental import pallas as pl
from jax.experimental.pallas import tpu as pltpu
from jax.experimental.pallas import tpu_sc as plsc

B, L, D, V = 4096, 50, 128, 1000000
N = B * L
LP = 56
NPAD = B * LP
EPS = 1e-12

NC, NS = 2, 16
NW = NC * NS
PER_W = NPAD // NW
CHUNK = 128
NCHUNK = PER_W // CHUNK


def _gather_body(idx_hbm, table_hbm, out_hbm, idx_v, rows_v, sem):
    wid = lax.axis_index("s") * NC + lax.axis_index("c")
    base = wid * PER_W
    pltpu.sync_copy(idx_hbm.at[pl.ds(base, PER_W)], idx_v)

    pltpu.async_copy(
        table_hbm.at[idx_v.at[pl.ds(0, CHUNK)]], rows_v.at[0], sem
    )

    def body(j, carry):
        cur = j % 2
        nxt = (j + 1) % 2
        pltpu.make_async_copy(
            table_hbm.at[idx_v.at[pl.ds(j * CHUNK, CHUNK)]], rows_v.at[cur],
            sem,
        ).wait()

        @pl.when(j + 1 < NCHUNK)
        def _start_next():
            pltpu.async_copy(
                table_hbm.at[idx_v.at[pl.ds((j + 1) * CHUNK, CHUNK)]],
                rows_v.at[nxt], sem,
            )

        pltpu.sync_copy(
            rows_v.at[cur], out_hbm.at[pl.ds(base + j * CHUNK, CHUNK)]
        )
        return carry

    lax.fori_loop(0, NCHUNK, body, 0)


@functools.cache
def _make_gather():
    return pl.kernel(
        _gather_body,
        mesh=plsc.VectorSubcoreMesh(core_axis_name="c", subcore_axis_name="s"),
        out_type=jax.ShapeDtypeStruct((NPAD, D), jnp.float32),
        scratch_types=[
            pltpu.VMEM((PER_W,), jnp.int32),
            pltpu.VMEM((2, CHUNK, D), jnp.float32),
            pltpu.SemaphoreType.DMA,
        ],
        compiler_params=pltpu.CompilerParams(use_tc_tiling_on_sc=True),
    )


BB = 64
TBP = BB * LP


def _tc_body(g_ref, ft_ref, wc_ref, bb_ref, gm_ref, bt_ref, o_ref):
    proj = jnp.dot(
        ft_ref[...].T, wc_ref[...], preferred_element_type=jnp.float32
    )
    emb = g_ref[...] + proj + bb_ref[...]
    mu = jnp.mean(emb, axis=-1, keepdims=True)
    dev = emb - mu
    var = jnp.mean(dev * dev, axis=-1, keepdims=True)
    res = dev * lax.rsqrt(var + EPS) * gm_ref[...] + bt_ref[...]
    o_ref[...] = res.reshape(BB, LP, D)[:, :L, :]


def _tc_call(gathered, feat, w_cat, bb, gm, bt):
    grid = (B // BB,)
    return pl.pallas_call(
        _tc_body,
        grid=grid,
        in_specs=[
            pl.BlockSpec((TBP, D), lambda i: (i, 0)),
            pl.BlockSpec((5, TBP), lambda i: (0, i)),
            pl.BlockSpec((5, D), lambda i: (0, 0)),
            pl.BlockSpec((1, D), lambda i: (0, 0)),
            pl.BlockSpec((1, D), lambda i: (0, 0)),
            pl.BlockSpec((1, D), lambda i: (0, 0)),
        ],
        out_specs=pl.BlockSpec((BB, L, D), lambda i: (i, 0, 0)),
        out_shape=jax.ShapeDtypeStruct((B, L, D), jnp.float32),
    )(gathered, feat, w_cat, bb, gm, bt)


def kernel(categories, boxes, scores, table, W_box, b_box, W_score, b_score,
           gamma, beta):
    idx = jnp.pad(
        categories.astype(jnp.int32), ((0, 0), (0, LP - L))
    ).reshape(NPAD)
    gathered = _make_gather()(idx, table)
    fcat = jnp.concatenate(
        [boxes, scores[..., None]], axis=-1
    )
    feat = jnp.pad(
        jnp.transpose(fcat, (2, 0, 1)), ((0, 0), (0, 0), (0, LP - L))
    ).reshape(5, NPAD)
    w_cat = jnp.concatenate([W_box, W_score], axis=0)
    bias = (b_box + b_score).reshape(1, D)
    return _tc_call(
        gathered,
        feat,
        w_cat,
        bias,
        gamma.reshape(1, D),
        beta.reshape(1, D),
    )


# speedup vs baseline: 4.2839x; 4.2839x over previous
import functools

import jax
import jax.numpy as jnp
from jax import lax
from jax.experimental import pallas as pl
from jax.experimental.pallas import tpu as pltpu
from jax.experimental.pallas import tpu_sc as plsc

B, L, D, V = 4096, 50, 128, 1000000
N = B * L
LP = 56
NPAD = B * LP
EPS = 1e-12

NC, NS = 2, 16
NW = NC * NS
PER_W = NPAD // NW
CHUNK = 128
NCHUNK = PER_W // CHUNK


def _gather_body(idx_hbm, table_hbm, out_hbm, idx_v, rows_v, sem):
    wid = lax.axis_index("s") * NC + lax.axis_index("c")
    base = wid * PER_W
    pltpu.sync_copy(idx_hbm.at[pl.ds(base, PER_W)], idx_v)

    pltpu.async_copy(
        table_hbm.at[idx_v.at[pl.ds(0, CHUNK)]], rows_v.at[0], sem
    )

    def body(j, carry):
        cur = j % 2
        nxt = (j + 1) % 2
        pltpu.make_async_copy(
            table_hbm.at[idx_v.at[pl.ds(j * CHUNK, CHUNK)]], rows_v.at[cur],
            sem,
        ).wait()

        @pl.when(j + 1 < NCHUNK)
        def _start_next():
            pltpu.async_copy(
                table_hbm.at[idx_v.at[pl.ds((j + 1) * CHUNK, CHUNK)]],
                rows_v.at[nxt], sem,
            )

        pltpu.sync_copy(
            rows_v.at[cur], out_hbm.at[pl.ds(base + j * CHUNK, CHUNK)]
        )
        return carry

    lax.fori_loop(0, NCHUNK, body, 0)


@functools.cache
def _make_gather():
    return pl.kernel(
        _gather_body,
        mesh=plsc.VectorSubcoreMesh(core_axis_name="c", subcore_axis_name="s"),
        out_type=jax.ShapeDtypeStruct((NPAD, D), jnp.float32),
        scratch_types=[
            pltpu.VMEM((PER_W,), jnp.int32),
            pltpu.VMEM((2, CHUNK, D), jnp.float32),
            pltpu.SemaphoreType.DMA,
        ],
        compiler_params=pltpu.CompilerParams(use_tc_tiling_on_sc=True),
    )


BB = 64
TBP = BB * LP


def _tc_body(g_ref, ft_ref, wc_ref, bb_ref, gm_ref, bt_ref, o_ref):
    proj = jnp.dot(
        ft_ref[...].T, wc_ref[...], preferred_element_type=jnp.float32
    )
    emb = g_ref[...] + proj + bb_ref[...]
    mu = jnp.mean(emb, axis=-1, keepdims=True)
    dev = emb - mu
    var = jnp.mean(dev * dev, axis=-1, keepdims=True)
    res = dev * lax.rsqrt(var + EPS) * gm_ref[...] + bt_ref[...]
    o_ref[...] = res.reshape(BB, LP, D)[:, :L, :]


def _tc_call(gathered, feat, w_cat, bb, gm, bt):
    grid = (B // BB,)
    return pl.pallas_call(
        _tc_body,
        grid=grid,
        in_specs=[
            pl.BlockSpec((TBP, D), lambda i: (i, 0)),
            pl.BlockSpec((5, TBP), lambda i: (0, i)),
            pl.BlockSpec((5, D), lambda i: (0, 0)),
            pl.BlockSpec((1, D), lambda i: (0, 0)),
            pl.BlockSpec((1, D), lambda i: (0, 0)),
            pl.BlockSpec((1, D), lambda i: (0, 0)),
        ],
        out_specs=pl.BlockSpec((BB, L, D), lambda i: (i, 0, 0)),
        out_shape=jax.ShapeDtypeStruct((B, L, D), jnp.float32),
    )(gathered, feat, w_cat, bb, gm, bt)


def kernel(categories, boxes, scores, table, W_box, b_box, W_score, b_score,
           gamma, beta):
    pos = lax.broadcasted_iota(jnp.int32, (B, LP), 0) * LP + lax.broadcasted_iota(
        jnp.int32, (B, LP), 1
    )
    idx = jnp.where(
        lax.broadcasted_iota(jnp.int32, (B, LP), 1) < L,
        jnp.pad(categories.astype(jnp.int32), ((0, 0), (0, LP - L))),
        pos % V,
    ).reshape(NPAD)
    gathered = _make_gather()(idx, table)
    fcat = jnp.concatenate(
        [boxes, scores[..., None]], axis=-1
    )
    feat = jnp.pad(
        jnp.transpose(fcat, (2, 0, 1)), ((0, 0), (0, 0), (0, LP - L))
    ).reshape(5, NPAD)
    w_cat = jnp.concatenate([W_box, W_score], axis=0)
    bias = (b_box + b_score).reshape(1, D)
    return _tc_call(
        gathered,
        feat,
        w_cat,
        bias,
        gamma.reshape(1, D),
        beta.reshape(1, D),
    )


# speedup vs baseline: 5.6937x; 1.3291x over previous
import functools

import jax
import jax.numpy as jnp
from jax import lax
from jax.experimental import pallas as pl
from jax.experimental.pallas import tpu as pltpu
from jax.experimental.pallas import tpu_sc as plsc

B, L, D, V = 4096, 50, 128, 1000000
N = B * L
EPS = 1e-12

NC, NS = 2, 16
NW = NC * NS
PER_W = N // NW
CHUNK = 128
NCHUNK = PER_W // CHUNK


def _gather_body(idx_hbm, table_hbm, out_hbm, idx_v, rows_v, sem):
    wid = lax.axis_index("s") * NC + lax.axis_index("c")
    base = wid * PER_W
    pltpu.sync_copy(idx_hbm.at[pl.ds(base, PER_W)], idx_v)

    pltpu.async_copy(
        table_hbm.at[idx_v.at[pl.ds(0, CHUNK)]], rows_v.at[0], sem
    )

    def body(j, carry):
        cur = j % 2
        nxt = (j + 1) % 2
        pltpu.make_async_copy(
            table_hbm.at[idx_v.at[pl.ds(j * CHUNK, CHUNK)]], rows_v.at[cur],
            sem,
        ).wait()

        @pl.when(j + 1 < NCHUNK)
        def _start_next():
            pltpu.async_copy(
                table_hbm.at[idx_v.at[pl.ds((j + 1) * CHUNK, CHUNK)]],
                rows_v.at[nxt], sem,
            )

        pltpu.sync_copy(
            rows_v.at[cur], out_hbm.at[pl.ds(base + j * CHUNK, CHUNK)]
        )
        return carry

    lax.fori_loop(0, NCHUNK, body, 0)


@functools.cache
def _make_gather():
    return pl.kernel(
        _gather_body,
        mesh=plsc.VectorSubcoreMesh(core_axis_name="c", subcore_axis_name="s"),
        out_type=jax.ShapeDtypeStruct((N, D), jnp.float32),
        scratch_types=[
            pltpu.VMEM((PER_W,), jnp.int32),
            pltpu.VMEM((2, CHUNK, D), jnp.float32),
            pltpu.SemaphoreType.DMA,
        ],
        compiler_params=pltpu.CompilerParams(use_tc_tiling_on_sc=True),
    )


TB = 2048


def _tc_body(g_ref, ft_ref, wc_ref, bb_ref, gm_ref, bt_ref, o_ref):
    proj = jnp.dot(
        ft_ref[...].T, wc_ref[...], preferred_element_type=jnp.float32
    )
    emb = g_ref[...] + proj + bb_ref[...]
    mu = jnp.mean(emb, axis=-1, keepdims=True)
    dev = emb - mu
    var = jnp.mean(dev * dev, axis=-1, keepdims=True)
    o_ref[...] = dev * lax.rsqrt(var + EPS) * gm_ref[...] + bt_ref[...]


def _tc_call(gathered, feat, w_cat, bb, gm, bt):
    grid = (N // TB,)
    return pl.pallas_call(
        _tc_body,
        grid=grid,
        in_specs=[
            pl.BlockSpec((TB, D), lambda i: (i, 0)),
            pl.BlockSpec((5, TB), lambda i: (0, i)),
            pl.BlockSpec((5, D), lambda i: (0, 0)),
            pl.BlockSpec((1, D), lambda i: (0, 0)),
            pl.BlockSpec((1, D), lambda i: (0, 0)),
            pl.BlockSpec((1, D), lambda i: (0, 0)),
        ],
        out_specs=pl.BlockSpec((TB, D), lambda i: (i, 0)),
        out_shape=jax.ShapeDtypeStruct((N, D), jnp.float32),
    )(gathered, feat, w_cat, bb, gm, bt)


def kernel(categories, boxes, scores, table, W_box, b_box, W_score, b_score,
           gamma, beta):
    idx = categories.astype(jnp.int32).T.reshape(N)
    gathered = _make_gather()(idx, table)
    feat = jnp.concatenate(
        [jnp.transpose(boxes, (2, 1, 0)), scores.T[None]], axis=0
    ).reshape(5, N)
    w_cat = jnp.concatenate([W_box, W_score], axis=0)
    bias = (b_box + b_score).reshape(1, D)
    out = _tc_call(
        gathered,
        feat,
        w_cat,
        bias,
        gamma.reshape(1, D),
        beta.reshape(1, D),
    )
    return jnp.transpose(out.reshape(L, B, D), (1, 0, 2))
